# baseline (speedup 1.0000x reference)
import functools

import jax
import jax.numpy as jnp
from jax import lax
from jax.experimental import pallas as pl
from jax.experimental.pallas import tpu as pltpu
from jax.experimental.pallas import tpu_sc as plsc

_B = 1024
_D = 64
_N = 100000
_K = 20
_KPAD = 128
_NT = 2048
_T = (_N + _NT - 1) // _NT
_NEG = float("-inf")


def _proj_body(ego_ref, wt_ref, bt_ref, zego_ref, qn_ref):
    z = jnp.dot(ego_ref[...], wt_ref[...], preferred_element_type=jnp.float32)
    z = z + bt_ref[...]
    zego_ref[...] = z
    nrm = jnp.sqrt(jnp.sum(z * z, axis=1, keepdims=True))
    qn_ref[...] = z / (nrm + 1e-6)


def _project(ego, W_t, b_t2):
    return pl.pallas_call(
        _proj_body,
        out_shape=(
            jax.ShapeDtypeStruct((_B, _D), jnp.float32),
            jax.ShapeDtypeStruct((_B, _D), jnp.float32),
        ),
    )(ego, W_t, b_t2)


_KS = 32
_NC = 16
_CW = _NT // _NC
_NLAST = _N - (_T - 1) * _NT


def _knn_body(qnT_ref, bank_ref, idx_ref, st_ref, vals_ref, gidx_ref,
              mg_ref, ag_ref, ex_ref, act_ref, iact_ref):
    t = pl.program_id(0)
    ln32 = lax.broadcasted_iota(jnp.int32, (_KS, _B), 0)
    ln16 = lax.broadcasted_iota(jnp.int32, (_NC, _B), 0)
    l128 = lax.broadcasted_iota(jnp.int32, (_CW, _B), 0)
    big = jnp.int32(2**30)

    @pl.when(t == 0)
    def _init():
        vals_ref[...] = jnp.where(ln32 < _K, _NEG, jnp.inf)
        gidx_ref[...] = jnp.zeros((_KS, _B), jnp.int32)

    bk = bank_ref[...]
    nrm = jnp.sqrt(jnp.sum(bk * bk, axis=1, keepdims=True))
    bn = bk / (nrm + 1e-6)
    sT = jnp.dot(bn, qnT_ref[...],
                 preferred_element_type=jnp.float32)

    @pl.when(t < _T - 1)
    def _store_full():
        st_ref[...] = sT

    @pl.when(t == _T - 1)
    def _store_masked():
        rowi = lax.broadcasted_iota(jnp.int32, (_NT, _B), 0)
        st_ref[...] = jnp.where(rowi < _NLAST, sT, _NEG)

    def _chunk_scan():
        mgs, ags = [], []
        for c in range(_NC):
            sc = st_ref[c * _CW:(c + 1) * _CW, :]
            mgc = jnp.max(sc, axis=0, keepdims=True)
            ags.append(jnp.min(jnp.where(sc == mgc, l128, big),
                               axis=0, keepdims=True))
            mgs.append(mgc)
        return jnp.concatenate(mgs, axis=0), jnp.concatenate(ags, axis=0)

    mg0, ag0 = _chunk_scan()
    mg_ref[...] = mg0
    ag_ref[...] = ag0
    bmin0 = jnp.min(vals_ref[...], axis=0, keepdims=True)
    act_ref[0] = jnp.any(mg0 > bmin0).astype(jnp.int32)

    def _inner(_, carry):
        @pl.when(iact_ref[0] == 1)
        def _extract():
            mgw = mg_ref[...]
            m = jnp.max(mgw, axis=0, keepdims=True)
            c = jnp.min(jnp.where(mgw == m, ln16, big), axis=0, keepdims=True)
            vals = vals_ref[...]
            bmin = jnp.min(vals, axis=0, keepdims=True)
            upd = m > bmin
            bpos = jnp.min(jnp.where(vals == bmin, ln32, big),
                           axis=0, keepdims=True)
            sel = (ln32 == bpos) & upd
            agv = jnp.sum(jnp.where(ln16 == c, ag_ref[...], 0),
                          axis=0, keepdims=True)
            gidx = t * _NT + c * _CW + agv
            vals_ref[...] = jnp.where(sel, m, vals)
            gidx_ref[...] = jnp.where(sel, gidx, gidx_ref[...])
            hit = (ln16 == c) & upd
            mgw2 = jnp.where(hit, _NEG, mgw)
            mg_ref[...] = mgw2
            ex_ref[...] = jnp.where(hit, 1, ex_ref[...])
            m2 = jnp.max(mgw2, axis=0, keepdims=True)
            bmin2 = jnp.min(vals_ref[...], axis=0, keepdims=True)
            iact_ref[0] = jnp.any(m2 > bmin2).astype(jnp.int32)
        return carry

    def _round(_, carry):
        @pl.when(act_ref[0] == 1)
        def _do():
            ex_ref[...] = jnp.zeros((_NC, _B), jnp.int32)
            iact_ref[0] = 1
            lax.fori_loop(0, _K, _inner, 0)
            ex = ex_ref[...]
            ag = ag_ref[...]
            mgs, ags = [], []
            for c in range(_NC):
                sc = st_ref[c * _CW:(c + 1) * _CW, :]
                kill = (l128 == ag[c:c + 1, :]) & (ex[c:c + 1, :] == 1)
                sc2 = jnp.where(kill, _NEG, sc)
                st_ref[c * _CW:(c + 1) * _CW, :] = sc2
                mgc = jnp.max(sc2, axis=0, keepdims=True)
                ags.append(jnp.min(jnp.where(sc2 == mgc, l128, big),
                                   axis=0, keepdims=True))
                mgs.append(mgc)
            mg2 = jnp.concatenate(mgs, axis=0)
            mg_ref[...] = mg2
            ag_ref[...] = jnp.concatenate(ags, axis=0)
            bmin3 = jnp.min(vals_ref[...], axis=0, keepdims=True)
            act_ref[0] = jnp.any(mg2 > bmin3).astype(jnp.int32)
        return carry

    lax.fori_loop(0, _K, _round, 0)

    @pl.when(t == _T - 1)
    def _emit():
        idx_ref[...] = gidx_ref[...]


def _knn_topk(qnT, bank):
    return pl.pallas_call(
        _knn_body,
        grid=(_T,),
        in_specs=[
            pl.BlockSpec((_D, _B), lambda t: (0, 0)),
            pl.BlockSpec((_NT, _D), lambda t: (t, 0)),
        ],
        out_specs=pl.BlockSpec((_KS, _B), lambda t: (0, 0)),
        out_shape=jax.ShapeDtypeStruct((_KS, _B), jnp.int32),
        scratch_shapes=[
            pltpu.VMEM((_NT, _B), jnp.float32),
            pltpu.VMEM((_KS, _B), jnp.float32),
            pltpu.VMEM((_KS, _B), jnp.int32),
            pltpu.VMEM((_NC, _B), jnp.float32),
            pltpu.VMEM((_NC, _B), jnp.int32),
            pltpu.VMEM((_NC, _B), jnp.int32),
            pltpu.SMEM((1,), jnp.int32),
            pltpu.SMEM((1,), jnp.int32),
        ],
    )(qnT, bank)


_NROWS = _B * _K
_NW = 32
_RPW = _NROWS // _NW


def _sc_gather(idx_flat, table):
    mesh = plsc.VectorSubcoreMesh(core_axis_name="c", subcore_axis_name="s")

    @functools.partial(
        pl.kernel,
        mesh=mesh,
        out_type=jax.ShapeDtypeStruct((_NROWS, _D), jnp.float32),
        scratch_types=[
            pltpu.VMEM((_RPW,), jnp.int32),
            pltpu.VMEM((_RPW, _D), jnp.float32),
            pltpu.SemaphoreType.DMA,
        ],
        compiler_params=pltpu.CompilerParams(use_tc_tiling_on_sc=False),
    )
    def _gk(idx_hbm, table_hbm, out_hbm, idx_v, rows_v, sem):
        wid = lax.axis_index("s") * 2 + lax.axis_index("c")
        base = wid * _RPW
        pltpu.sync_copy(idx_hbm.at[pl.ds(base, _RPW)], idx_v)
        pltpu.async_copy(table_hbm.at[idx_v], rows_v, sem).wait()
        pltpu.sync_copy(rows_v, out_hbm.at[pl.ds(base, _RPW)])

    return _gk(idx_flat, table)


_BS = 256


def _attn_body(zego_ref, zexo_ref, wq_ref, wk_ref, wv_ref, wo_ref,
               wcv_ref, bcv_ref, wcn_ref, bcn_ref, mask_ref,
               z_ref, verb_ref, noun_ref):
    z_e = zego_ref[...]
    ze2 = zexo_ref[...]
    q = jnp.dot(z_e, wq_ref[...], preferred_element_type=jnp.float32)
    kk = jnp.dot(ze2, wk_ref[...], preferred_element_type=jnp.float32)
    vv = jnp.dot(ze2, wv_ref[...], preferred_element_type=jnp.float32)
    kk3 = kk.reshape(_BS, _K, _D)
    l = jnp.sum(q[:, None, :] * kk3, axis=2) * (1.0 / 8.0)
    l = l + mask_ref[...]
    l = l - jnp.max(l, axis=1, keepdims=True)
    p = jnp.exp(l)
    p = p / jnp.sum(p, axis=1, keepdims=True)
    vv3 = vv.reshape(_BS, _K, _D)
    ctx = jnp.sum(p[:, :, None] * vv3, axis=1)
    z = z_e + jnp.dot(ctx, wo_ref[...], preferred_element_type=jnp.float32)
    z_ref[...] = z
    verb_ref[...] = jnp.dot(z, wcv_ref[...],
                            preferred_element_type=jnp.float32) + bcv_ref[...]
    noun_ref[...] = jnp.dot(z, wcn_ref[...],
                            preferred_element_type=jnp.float32) + bcn_ref[...]


def _attn_heads(zego, zexo_flat, W_q, W_k, W_v, W_o, W_cv, b_cv2, W_cn,
                b_cn2, maskadd):
    nv = W_cv.shape[1]
    nn = W_cn.shape[1]
    grid = (_B // _BS,)
    wspec = pl.BlockSpec((_D, _D), lambda i: (0, 0))
    return pl.pallas_call(
        _attn_body,
        grid=grid,
        in_specs=[
            pl.BlockSpec((_BS, _D), lambda i: (i, 0)),
            pl.BlockSpec((_BS * _K, _D), lambda i: (i, 0)),
            wspec, wspec, wspec, wspec,
            pl.BlockSpec((_D, nv), lambda i: (0, 0)),
            pl.BlockSpec((1, nv), lambda i: (0, 0)),
            pl.BlockSpec((_D, nn), lambda i: (0, 0)),
            pl.BlockSpec((1, nn), lambda i: (0, 0)),
            pl.BlockSpec((1, _K), lambda i: (0, 0)),
        ],
        out_specs=(
            pl.BlockSpec((_BS, _D), lambda i: (i, 0)),
            pl.BlockSpec((_BS, nv), lambda i: (i, 0)),
            pl.BlockSpec((_BS, nn), lambda i: (i, 0)),
        ),
        out_shape=(
            jax.ShapeDtypeStruct((_B, _D), jnp.float32),
            jax.ShapeDtypeStruct((_B, nv), jnp.float32),
            jax.ShapeDtypeStruct((_B, nn), jnp.float32),
        ),
    )(zego, zexo_flat, W_q, W_k, W_v, W_o, W_cv, b_cv2, W_cn, b_cn2, maskadd)


def kernel(ego_input, exo_bank, W_t, b_t, W_q, W_k, W_v, W_o, W_cv, b_cv,
           W_cn, b_cn, k, k_active):
    zego, qn = _project(ego_input, W_t, b_t.reshape(1, _D))
    idx_pad = _knn_topk(qn.T, exo_bank)
    idx_flat = idx_pad[:_K, :].T.reshape(_NROWS)
    zexo_flat = _sc_gather(idx_flat, exo_bank)
    maskadd = jnp.where(jnp.arange(_K)[None, :] < k_active,
                        0.0, -1e9).astype(jnp.float32)
    z, verb, noun = _attn_heads(zego, zexo_flat, W_q, W_k, W_v, W_o,
                                W_cv, b_cv.reshape(1, -1),
                                W_cn, b_cn.reshape(1, -1), maskadd)
    return (z, verb, noun)


# speedup vs baseline: 1.0853x; 1.0853x over previous
import functools

import jax
import jax.numpy as jnp
from jax import lax
from jax.experimental import pallas as pl
from jax.experimental.pallas import tpu as pltpu
from jax.experimental.pallas import tpu_sc as plsc

_B = 1024
_D = 64
_N = 100000
_K = 20
_KPAD = 128
_NT = 2048
_T = (_N + _NT - 1) // _NT
_NEG = float("-inf")


def _proj_body(ego_ref, wt_ref, bt_ref, zego_ref, qn_ref):
    z = jnp.dot(ego_ref[...], wt_ref[...], preferred_element_type=jnp.float32)
    z = z + bt_ref[...]
    zego_ref[...] = z
    nrm = jnp.sqrt(jnp.sum(z * z, axis=1, keepdims=True))
    qn_ref[...] = z / (nrm + 1e-6)


def _project(ego, W_t, b_t2):
    return pl.pallas_call(
        _proj_body,
        out_shape=(
            jax.ShapeDtypeStruct((_B, _D), jnp.float32),
            jax.ShapeDtypeStruct((_B, _D), jnp.float32),
        ),
    )(ego, W_t, b_t2)


_KS = 32
_NC = 16
_CW = _NT // _NC
_NLAST = _N - (_T - 1) * _NT


def _knn_body(qnT_ref, bank_ref, idx_ref, st_ref, vals_ref, gidx_ref,
              mg_ref, ag_ref, ex_ref, act_ref, iact_ref, flg_ref):
    t = pl.program_id(0)
    ln32 = lax.broadcasted_iota(jnp.int32, (_KS, _B), 0)
    ln16 = lax.broadcasted_iota(jnp.int32, (_NC, _B), 0)
    l128 = lax.broadcasted_iota(jnp.int32, (_CW, _B), 0)
    big = jnp.int32(2**30)

    @pl.when(t == 0)
    def _init():
        vals_ref[...] = jnp.where(ln32 < _K, _NEG, jnp.inf)
        gidx_ref[...] = jnp.zeros((_KS, _B), jnp.int32)

    bk = bank_ref[...]
    nrm = jnp.sqrt(jnp.sum(bk * bk, axis=1, keepdims=True))
    bn = bk / (nrm + 1e-6)
    sT = jnp.dot(bn, qnT_ref[...],
                 preferred_element_type=jnp.float32)

    @pl.when(t < _T - 1)
    def _store_full():
        st_ref[...] = sT

    @pl.when(t == _T - 1)
    def _store_masked():
        rowi = lax.broadcasted_iota(jnp.int32, (_NT, _B), 0)
        st_ref[...] = jnp.where(rowi < _NLAST, sT, _NEG)

    for c in range(_NC):
        sc = st_ref[c * _CW:(c + 1) * _CW, :]
        mgc = jnp.max(sc, axis=0, keepdims=True)
        mg_ref[c:c + 1, :] = mgc
        ag_ref[c:c + 1, :] = jnp.min(jnp.where(sc == mgc, l128, big),
                                     axis=0, keepdims=True)
    bmin0 = jnp.min(vals_ref[...], axis=0, keepdims=True)
    act_ref[0] = jnp.any(mg_ref[...] > bmin0).astype(jnp.int32)

    def _inner(_, carry):
        @pl.when(iact_ref[0] == 1)
        def _extract():
            mgw = mg_ref[...]
            m = jnp.max(mgw, axis=0, keepdims=True)
            c = jnp.min(jnp.where(mgw == m, ln16, big), axis=0, keepdims=True)
            vals = vals_ref[...]
            bmin = jnp.min(vals, axis=0, keepdims=True)
            upd = m > bmin
            bpos = jnp.min(jnp.where(vals == bmin, ln32, big),
                           axis=0, keepdims=True)
            sel = (ln32 == bpos) & upd
            agv = jnp.sum(jnp.where(ln16 == c, ag_ref[...], 0),
                          axis=0, keepdims=True)
            gidx = t * _NT + c * _CW + agv
            vals_ref[...] = jnp.where(sel, m, vals)
            gidx_ref[...] = jnp.where(sel, gidx, gidx_ref[...])
            hit = (ln16 == c) & upd
            mgw2 = jnp.where(hit, _NEG, mgw)
            mg_ref[...] = mgw2
            ex_ref[...] = jnp.where(hit, 1, ex_ref[...])
            m2 = jnp.max(mgw2, axis=0, keepdims=True)
            bmin2 = jnp.min(vals_ref[...], axis=0, keepdims=True)
            iact_ref[0] = jnp.any(m2 > bmin2).astype(jnp.int32)
        return carry

    def _round(_, carry):
        @pl.when(act_ref[0] == 1)
        def _do():
            ex_ref[...] = jnp.zeros((_NC, _B), jnp.int32)
            iact_ref[0] = 1
            lax.fori_loop(0, _K, _inner, 0)
            ex = ex_ref[...]
            ag = ag_ref[...]
            for c in range(_NC):
                flg_ref[c] = jnp.max(ex[c:c + 1, :])

            for c in range(_NC):
                @pl.when(flg_ref[c] == 1)
                def _fix(c=c):
                    sc = st_ref[c * _CW:(c + 1) * _CW, :]
                    kill = (l128 == ag[c:c + 1, :]) & (ex[c:c + 1, :] == 1)
                    sc2 = jnp.where(kill, _NEG, sc)
                    st_ref[c * _CW:(c + 1) * _CW, :] = sc2
                    mgc = jnp.max(sc2, axis=0, keepdims=True)
                    mg_ref[c:c + 1, :] = mgc
                    ag_ref[c:c + 1, :] = jnp.min(
                        jnp.where(sc2 == mgc, l128, big), axis=0, keepdims=True)
            bmin3 = jnp.min(vals_ref[...], axis=0, keepdims=True)
            act_ref[0] = jnp.any(mg_ref[...] > bmin3).astype(jnp.int32)
        return carry

    lax.fori_loop(0, _K, _round, 0)

    @pl.when(t == _T - 1)
    def _emit():
        idx_ref[...] = gidx_ref[...]


def _knn_topk(qnT, bank):
    return pl.pallas_call(
        _knn_body,
        grid=(_T,),
        in_specs=[
            pl.BlockSpec((_D, _B), lambda t: (0, 0)),
            pl.BlockSpec((_NT, _D), lambda t: (t, 0)),
        ],
        out_specs=pl.BlockSpec((_KS, _B), lambda t: (0, 0)),
        out_shape=jax.ShapeDtypeStruct((_KS, _B), jnp.int32),
        scratch_shapes=[
            pltpu.VMEM((_NT, _B), jnp.float32),
            pltpu.VMEM((_KS, _B), jnp.float32),
            pltpu.VMEM((_KS, _B), jnp.int32),
            pltpu.VMEM((_NC, _B), jnp.float32),
            pltpu.VMEM((_NC, _B), jnp.int32),
            pltpu.VMEM((_NC, _B), jnp.int32),
            pltpu.SMEM((1,), jnp.int32),
            pltpu.SMEM((1,), jnp.int32),
            pltpu.SMEM((_NC,), jnp.int32),
        ],
    )(qnT, bank)


_NROWS = _B * _K
_NW = 32
_RPW = _NROWS // _NW


def _sc_gather(idx_flat, table):
    mesh = plsc.VectorSubcoreMesh(core_axis_name="c", subcore_axis_name="s")

    @functools.partial(
        pl.kernel,
        mesh=mesh,
        out_type=jax.ShapeDtypeStruct((_NROWS, _D), jnp.float32),
        scratch_types=[
            pltpu.VMEM((_RPW,), jnp.int32),
            pltpu.VMEM((_RPW, _D), jnp.float32),
            pltpu.SemaphoreType.DMA,
        ],
        compiler_params=pltpu.CompilerParams(use_tc_tiling_on_sc=False),
    )
    def _gk(idx_hbm, table_hbm, out_hbm, idx_v, rows_v, sem):
        wid = lax.axis_index("s") * 2 + lax.axis_index("c")
        base = wid * _RPW
        pltpu.sync_copy(idx_hbm.at[pl.ds(base, _RPW)], idx_v)
        pltpu.async_copy(table_hbm.at[idx_v], rows_v, sem).wait()
        pltpu.sync_copy(rows_v, out_hbm.at[pl.ds(base, _RPW)])

    return _gk(idx_flat, table)


_BS = 256


def _attn_body(zego_ref, zexo_ref, wq_ref, wk_ref, wv_ref, wo_ref,
               wcv_ref, bcv_ref, wcn_ref, bcn_ref, mask_ref,
               z_ref, verb_ref, noun_ref):
    z_e = zego_ref[...]
    ze2 = zexo_ref[...]
    q = jnp.dot(z_e, wq_ref[...], preferred_element_type=jnp.float32)
    kk = jnp.dot(ze2, wk_ref[...], preferred_element_type=jnp.float32)
    vv = jnp.dot(ze2, wv_ref[...], preferred_element_type=jnp.float32)
    kk3 = kk.reshape(_BS, _K, _D)
    l = jnp.sum(q[:, None, :] * kk3, axis=2) * (1.0 / 8.0)
    l = l + mask_ref[...]
    l = l - jnp.max(l, axis=1, keepdims=True)
    p = jnp.exp(l)
    p = p / jnp.sum(p, axis=1, keepdims=True)
    vv3 = vv.reshape(_BS, _K, _D)
    ctx = jnp.sum(p[:, :, None] * vv3, axis=1)
    z = z_e + jnp.dot(ctx, wo_ref[...], preferred_element_type=jnp.float32)
    z_ref[...] = z
    verb_ref[...] = jnp.dot(z, wcv_ref[...],
                            preferred_element_type=jnp.float32) + bcv_ref[...]
    noun_ref[...] = jnp.dot(z, wcn_ref[...],
                            preferred_element_type=jnp.float32) + bcn_ref[...]


def _attn_heads(zego, zexo_flat, W_q, W_k, W_v, W_o, W_cv, b_cv2, W_cn,
                b_cn2, maskadd):
    nv = W_cv.shape[1]
    nn = W_cn.shape[1]
    grid = (_B // _BS,)
    wspec = pl.BlockSpec((_D, _D), lambda i: (0, 0))
    return pl.pallas_call(
        _attn_body,
        grid=grid,
        in_specs=[
            pl.BlockSpec((_BS, _D), lambda i: (i, 0)),
            pl.BlockSpec((_BS * _K, _D), lambda i: (i, 0)),
            wspec, wspec, wspec, wspec,
            pl.BlockSpec((_D, nv), lambda i: (0, 0)),
            pl.BlockSpec((1, nv), lambda i: (0, 0)),
            pl.BlockSpec((_D, nn), lambda i: (0, 0)),
            pl.BlockSpec((1, nn), lambda i: (0, 0)),
            pl.BlockSpec((1, _K), lambda i: (0, 0)),
        ],
        out_specs=(
            pl.BlockSpec((_BS, _D), lambda i: (i, 0)),
            pl.BlockSpec((_BS, nv), lambda i: (i, 0)),
            pl.BlockSpec((_BS, nn), lambda i: (i, 0)),
        ),
        out_shape=(
            jax.ShapeDtypeStruct((_B, _D), jnp.float32),
            jax.ShapeDtypeStruct((_B, nv), jnp.float32),
            jax.ShapeDtypeStruct((_B, nn), jnp.float32),
        ),
    )(zego, zexo_flat, W_q, W_k, W_v, W_o, W_cv, b_cv2, W_cn, b_cn2, maskadd)


def kernel(ego_input, exo_bank, W_t, b_t, W_q, W_k, W_v, W_o, W_cv, b_cv,
           W_cn, b_cn, k, k_active):
    zego, qn = _project(ego_input, W_t, b_t.reshape(1, _D))
    idx_pad = _knn_topk(qn.T, exo_bank)
    idx_flat = idx_pad[:_K, :].T.reshape(_NROWS)
    zexo_flat = _sc_gather(idx_flat, exo_bank)
    maskadd = jnp.where(jnp.arange(_K)[None, :] < k_active,
                        0.0, -1e9).astype(jnp.float32)
    z, verb, noun = _attn_heads(zego, zexo_flat, W_q, W_k, W_v, W_o,
                                W_cv, b_cv.reshape(1, -1),
                                W_cn, b_cn.reshape(1, -1), maskadd)
    return (z, verb, noun)
